# Initial kernel scaffold; baseline (speedup 1.0000x reference)
import jax
import jax.numpy as jnp
from jax.experimental import pallas as pl


def kernel(x, W, gamma, beta):
    raise NotImplementedError("write your pallas kernel here")


# speedup vs baseline: 11.4407x; 11.4407x over previous
import functools

import jax
import jax.numpy as jnp
from jax import lax
from jax.experimental import pallas as pl
from jax.experimental.pallas import tpu as pltpu
from jax.experimental.pallas import tpu_sc as plsc

B, C, N, K, OUT = 4, 128, 4096, 20, 256
BN = B * N
KPAD = 24
R = 256
NEG_INF = float("-inf")

NC, NS = 2, 16
NW = NC * NS
RPW = BN // NW
CH = 32
NCH = RPW // CH


def _knn_body(xt_ref, xf_ref, wcat_ref, idx_ref, u_ref, v_ref):
    b = pl.program_id(0)
    xtile = xt_ref[0]
    xfull = xf_ref[0]
    uv = lax.dot_general(xtile, wcat_ref[...], (((0,), (0,)), ((), ())),
                         preferred_element_type=jnp.float32)
    u_ref[0] = uv[:, :OUT]
    v_ref[0] = uv[:, OUT:]

    inner = lax.dot_general(xtile, xfull, (((0,), (0,)), ((), ())),
                            preferred_element_type=jnp.float32)
    xx_col = jnp.sum(xfull * xfull, axis=0, keepdims=True)
    dist = 2.0 * inner - xx_col
    iota = lax.broadcasted_iota(jnp.int32, (R, N), 1)
    base = b * N
    am0 = None
    for kk in range(K):
        m = jnp.max(dist, axis=1, keepdims=True)
        cand = jnp.where(dist == m, iota, N)
        am = jnp.min(cand, axis=1, keepdims=True)
        idx_ref[0, :, kk:kk + 1] = am + base
        if am0 is None:
            am0 = am + base
        dist = jnp.where(iota == am, NEG_INF, dist)
    for kk in range(K, KPAD):
        idx_ref[0, :, kk:kk + 1] = am0


def _run_knn(x, wcat):
    return pl.pallas_call(
        _knn_body,
        grid=(B, N // R),
        in_specs=[
            pl.BlockSpec((1, C, R), lambda b, rb: (b, 0, rb)),
            pl.BlockSpec((1, C, N), lambda b, rb: (b, 0, 0)),
            pl.BlockSpec((C, 2 * OUT), lambda b, rb: (0, 0)),
        ],
        out_specs=[
            pl.BlockSpec((1, R, KPAD), lambda b, rb: (b, rb, 0)),
            pl.BlockSpec((1, R, OUT), lambda b, rb: (b, rb, 0)),
            pl.BlockSpec((1, R, OUT), lambda b, rb: (b, rb, 0)),
        ],
        out_shape=[
            jax.ShapeDtypeStruct((B, N, KPAD), jnp.int32),
            jax.ShapeDtypeStruct((B, N, OUT), jnp.float32),
            jax.ShapeDtypeStruct((B, N, OUT), jnp.float32),
        ],
    )(x, x, wcat)


def _sc_body(u_hbm, idx_hbm, s_hbm, q_hbm, mx_hbm, mn_hbm,
             idx_v, g_v, st_v, sem0, sem1, semd):
    cid = lax.axis_index("c")
    sid = lax.axis_index("s")
    wid = sid * NC + cid
    base = wid * RPW

    pltpu.sync_copy(idx_hbm.at[pl.ds(base * KPAD, RPW * KPAD)], idx_v)

    def fire(local_r, gbuf, sem):
        src = u_hbm.at[idx_v.at[pl.ds(local_r * KPAD, K)]]
        pltpu.async_copy(src, gbuf, sem)

    def reduce_row(bi, jc):
        def chunk(c, carry):
            off = c * 16
            g0 = g_v[bi, 0, pl.ds(off, 16)]
            accs, accq, accM, accm = g0, g0 * g0, g0, g0
            for k in range(1, K):
                gk = g_v[bi, k, pl.ds(off, 16)]
                accs = accs + gk
                accq = accq + gk * gk
                accM = jnp.maximum(accM, gk)
                accm = jnp.minimum(accm, gk)
            st_v[0, jc, pl.ds(off, 16)] = accs
            st_v[1, jc, pl.ds(off, 16)] = accq
            st_v[2, jc, pl.ds(off, 16)] = accM
            st_v[3, jc, pl.ds(off, 16)] = accm
            return carry
        lax.fori_loop(0, OUT // 16, chunk, 0)

    fire(0, g_v.at[0], sem0)
    fire(1, g_v.at[1], sem1)

    def outer(ch, carry):
        def pair(j2, carry2):
            r0 = ch * CH + 2 * j2
            pltpu.make_async_copy(u_hbm.at[idx_v.at[pl.ds(r0 * KPAD, K)]],
                                  g_v.at[0], sem0).wait()
            reduce_row(0, 2 * j2)

            @pl.when(r0 + 2 < RPW)
            def _():
                fire(r0 + 2, g_v.at[0], sem0)

            pltpu.make_async_copy(
                u_hbm.at[idx_v.at[pl.ds((r0 + 1) * KPAD, K)]],
                g_v.at[1], sem1).wait()
            reduce_row(1, 2 * j2 + 1)

            @pl.when(r0 + 3 < RPW)
            def _():
                fire(r0 + 3, g_v.at[1], sem1)
            return carry2
        lax.fori_loop(0, CH // 2, pair, 0)

        row0 = base + ch * CH
        cps = [
            pltpu.async_copy(st_v.at[0], s_hbm.at[pl.ds(row0, CH), :], semd),
            pltpu.async_copy(st_v.at[1], q_hbm.at[pl.ds(row0, CH), :], semd),
            pltpu.async_copy(st_v.at[2], mx_hbm.at[pl.ds(row0, CH), :], semd),
            pltpu.async_copy(st_v.at[3], mn_hbm.at[pl.ds(row0, CH), :], semd),
        ]
        for cp in cps:
            cp.wait()
        return carry
    lax.fori_loop(0, NCH, outer, 0)


def _run_sc(u2, idx2):
    mesh = plsc.VectorSubcoreMesh(core_axis_name="c", subcore_axis_name="s",
                                  num_cores=NC, num_subcores=NS)
    f = pl.kernel(
        _sc_body,
        out_type=[jax.ShapeDtypeStruct((BN, OUT), jnp.float32)] * 4,
        mesh=mesh,
        scratch_types=[
            pltpu.VMEM((RPW * KPAD,), jnp.int32),
            pltpu.VMEM((2, K, OUT), jnp.float32),
            pltpu.VMEM((4, CH, OUT), jnp.float32),
            pltpu.SemaphoreType.DMA,
            pltpu.SemaphoreType.DMA,
            pltpu.SemaphoreType.DMA,
        ],
    )
    return f(u2, idx2)


_C1_ROWS = 1024
_C1_STEPS = BN // _C1_ROWS


def _c1_body(s_ref, q_ref, v_ref, gam_ref, bet_ref, stats_ref, acc_ref):
    step = pl.program_id(0)

    @pl.when(step == 0)
    def _():
        acc_ref[...] = jnp.zeros_like(acc_ref)

    sb = s_ref[...]
    qb = q_ref[...]
    vb = v_ref[...]
    acc_ref[0:1] += jnp.sum(sb, axis=0, keepdims=True)
    acc_ref[1:2] += jnp.sum(qb, axis=0, keepdims=True)
    acc_ref[2:3] += jnp.sum(vb * sb, axis=0, keepdims=True)
    acc_ref[3:4] += jnp.sum(vb, axis=0, keepdims=True)
    acc_ref[4:5] += jnp.sum(vb * vb, axis=0, keepdims=True)

    @pl.when(step == _C1_STEPS - 1)
    def _():
        inv = jnp.float32(1.0 / (BN * K))
        t1 = acc_ref[0:1]
        t2 = acc_ref[1:2]
        p = acc_ref[2:3]
        sv = acc_ref[3:4]
        sv2 = acc_ref[4:5]
        mean = (t1 + K * sv) * inv
        ey2 = (t2 + 2.0 * p + K * sv2) * inv
        var = ey2 - mean * mean
        scale = gam_ref[...] * lax.rsqrt(var + 1e-5)
        shift = bet_ref[...] - mean * scale
        stats_ref[0:1] = scale
        stats_ref[1:2] = shift


def _run_c1(s2, q2, v2, gamma, beta):
    return pl.pallas_call(
        _c1_body,
        grid=(_C1_STEPS,),
        in_specs=[
            pl.BlockSpec((_C1_ROWS, OUT), lambda i: (i, 0)),
            pl.BlockSpec((_C1_ROWS, OUT), lambda i: (i, 0)),
            pl.BlockSpec((_C1_ROWS, OUT), lambda i: (i, 0)),
            pl.BlockSpec((1, OUT), lambda i: (0, 0)),
            pl.BlockSpec((1, OUT), lambda i: (0, 0)),
        ],
        out_specs=pl.BlockSpec((2, OUT), lambda i: (0, 0)),
        out_shape=jax.ShapeDtypeStruct((2, OUT), jnp.float32),
        scratch_shapes=[pltpu.VMEM((8, OUT), jnp.float32)],
    )(s2, q2, v2, gamma, beta)


_C2_T = 512


def _c2_body(mx_ref, mn_ref, v_ref, stats_ref, gam_ref, out_ref):
    mxb = mx_ref[0]
    mnb = mn_ref[0]
    vb = v_ref[0]
    scale = stats_ref[0:1]
    shift = stats_ref[1:2]
    sel = jnp.where(gam_ref[...] >= 0.0, mxb, mnb)
    z = (sel + vb) * scale + shift
    res = jnp.where(z >= 0.0, z, 0.2 * z)
    out_ref[0] = res.T


def _run_c2(mx3, mn3, v3, stats, gamma):
    return pl.pallas_call(
        _c2_body,
        grid=(B, N // _C2_T),
        in_specs=[
            pl.BlockSpec((1, _C2_T, OUT), lambda b, t: (b, t, 0)),
            pl.BlockSpec((1, _C2_T, OUT), lambda b, t: (b, t, 0)),
            pl.BlockSpec((1, _C2_T, OUT), lambda b, t: (b, t, 0)),
            pl.BlockSpec((2, OUT), lambda b, t: (0, 0)),
            pl.BlockSpec((1, OUT), lambda b, t: (0, 0)),
        ],
        out_specs=pl.BlockSpec((1, OUT, _C2_T), lambda b, t: (b, 0, t)),
        out_shape=jax.ShapeDtypeStruct((B, OUT, N), jnp.float32),
    )(mx3, mn3, v3, stats, gamma)


def kernel(x, W, gamma, beta):
    w1t = jnp.transpose(W[:, :C])
    w2t = jnp.transpose(W[:, C:])
    wcat = jnp.concatenate([w1t, w2t - w1t], axis=1)

    idx, u, v = _run_knn(x, wcat)
    u2 = u.reshape(BN, OUT)
    idx2 = idx.reshape(BN * KPAD)

    s2, q2, mx2, mn2 = _run_sc(u2, idx2)

    g2 = gamma.reshape(1, OUT)
    b2 = beta.reshape(1, OUT)
    stats = _run_c1(s2, q2, v.reshape(BN, OUT), g2, b2)

    return _run_c2(mx2.reshape(B, N, OUT), mn2.reshape(B, N, OUT),
                   v, stats, g2)
